# Initial kernel scaffold; baseline (speedup 1.0000x reference)
import jax
import jax.numpy as jnp
from jax.experimental import pallas as pl


def kernel(input_ids, embed_tokens_weight):
    raise NotImplementedError("write your pallas kernel here")


# speedup vs baseline: 1.7564x; 1.7564x over previous
import functools

import jax
import jax.numpy as jnp
from jax import lax
from jax.experimental import pallas as pl
from jax.experimental.pallas import tpu as pltpu
from jax.experimental.pallas import tpu_sc as plsc

_DIM = 2048
_B = 4 * 4096
_NC = 2
_NS = 16
_NW = _NC * _NS
_BPW = _B // _NW
_CHUNK = 16
_NCHUNK = _BPW // _CHUNK
_NBUF = 2

_mesh = plsc.VectorSubcoreMesh(core_axis_name="c", subcore_axis_name="s")


@functools.partial(
    pl.kernel,
    mesh=_mesh,
    out_type=jax.ShapeDtypeStruct((_B, _DIM), jnp.float32),
    scratch_types=[
        pltpu.VMEM((_NCHUNK, _CHUNK), jnp.int32),
        pltpu.VMEM((_CHUNK, _DIM), jnp.float32),
        pltpu.VMEM((_CHUNK, _DIM), jnp.float32),
        pltpu.SemaphoreType.DMA,
        pltpu.SemaphoreType.DMA,
        pltpu.SemaphoreType.DMA,
        pltpu.SemaphoreType.DMA,
    ],
)
def _embed_gather(idx_hbm, table_hbm, out_hbm, idx_v, rows0, rows1,
                  g0, g1, s0, s1):
    rows = (rows0, rows1)
    gsem = (g0, g1)
    ssem = (s0, s1)
    wid = lax.axis_index("s") * _NC + lax.axis_index("c")
    base = wid * _BPW

    pltpu.sync_copy(idx_hbm.at[wid], idx_v)

    for b in range(_NBUF):
        pltpu.make_async_copy(
            table_hbm.at[idx_v.at[b]], rows[b], gsem[b]).start()

    def body(j, carry):
        for b in range(_NBUF):
            jj = j * _NBUF + b
            pltpu.make_async_copy(
                table_hbm.at[idx_v.at[jj]], rows[b], gsem[b]).wait()
            pltpu.make_async_copy(
                rows[b],
                out_hbm.at[pl.ds(base + jj * _CHUNK, _CHUNK)],
                ssem[b]).start()

            @pl.when(jj + _NBUF < _NCHUNK)
            def _():
                pltpu.make_async_copy(
                    rows[b],
                    out_hbm.at[pl.ds(base, _CHUNK)],
                    ssem[b]).wait()
                pltpu.make_async_copy(
                    table_hbm.at[idx_v.at[jj + _NBUF]], rows[b],
                    gsem[b]).start()
        return carry

    lax.fori_loop(0, _NCHUNK // _NBUF, body, 0)

    for b in range(_NBUF):
        pltpu.make_async_copy(
            rows[b],
            out_hbm.at[pl.ds(base, _CHUNK)],
            ssem[b]).wait()


def kernel(input_ids, embed_tokens_weight):
    idx = input_ids.reshape(_NW, _NCHUNK, _CHUNK)
    out = _embed_gather(idx, embed_tokens_weight)
    return out.reshape(input_ids.shape + (_DIM,))
